# baseline (speedup 1.0000x reference)
import jax
import jax.numpy as jnp

L = 5


def kernel(x, e, edge_index, node_emb, edge_embs, W1s, b1s, W2s, b2s, gammas, betas,
           RW0, Rb0, RW1, Rb1, RW2, Rb2):
    N = x.shape[0]
    h = jnp.take(node_emb, x, axis=0)
    src = edge_index[0]
    dst = edge_index[1]
    for l in range(L):
        ee = jnp.take(edge_embs[l], e, axis=0)
        m = jnp.take(h, src, axis=0) + ee
        agg = jax.ops.segment_sum(m, dst, num_segments=N)
        z = agg @ W1s[l] + b1s[l]
        z = jax.nn.relu(z)
        z = z @ W2s[l] + b2s[l]
        mean = jnp.mean(z, axis=0)
        var = jnp.var(z, axis=0)
        z = (z - mean) / jnp.sqrt(var + 1e-5) * gammas[l] + betas[l]
        if l < L - 1:
            z = jax.nn.relu(z)
        h = z
    hg = jnp.mean(h, axis=0, keepdims=True)
    y = jax.nn.relu(hg @ RW0 + Rb0)
    y = jax.nn.relu(y @ RW1 + Rb1)
    scores = y @ RW2 + Rb2
    return hg, scores


# speedup vs baseline: 1.2057x; 1.2057x over previous
import functools

import jax
import jax.numpy as jnp
from jax import lax
from jax.experimental import pallas as pl
from jax.experimental.pallas import tpu as pltpu
from jax.experimental.pallas import tpu_sc as plsc

_L = 5
_D = 128
_NW = 32
_GC = 400


def _make_sc_gather(B, V):
    b_per_w = B // _NW
    chunk = b_per_w if b_per_w <= 512 else _GC
    niter = b_per_w // chunk
    mesh = plsc.VectorSubcoreMesh(core_axis_name="c", subcore_axis_name="s")

    @functools.partial(
        pl.kernel,
        mesh=mesh,
        out_type=jax.ShapeDtypeStruct((B, _D), jnp.float32),
        scratch_types=[
            pltpu.VMEM((b_per_w,), jnp.int32),
            pltpu.VMEM((chunk, _D), jnp.float32),
            pltpu.SemaphoreType.DMA,
        ],
    )
    def k(table_hbm, idx_hbm, out_hbm, idx_v, rows_v, gsem):
        wid = lax.axis_index("s") * 2 + lax.axis_index("c")
        base = wid * b_per_w
        pltpu.sync_copy(idx_hbm.at[pl.ds(base, b_per_w)], idx_v)

        def step(i, carry):
            pltpu.async_copy(
                table_hbm.at[idx_v.at[pl.ds(i * chunk, chunk)]], rows_v, gsem
            ).wait()
            pltpu.sync_copy(rows_v, out_hbm.at[pl.ds(base + i * chunk, chunk)])
            return carry

        lax.fori_loop(0, niter, step, 0, unroll=False)

    return k


def _sc_gather(table, idx):
    return _make_sc_gather(idx.shape[0], table.shape[0])(table, idx)


def kernel(x, e, edge_index, node_emb, edge_embs, W1s, b1s, W2s, b2s, gammas, betas,
           RW0, Rb0, RW1, Rb1, RW2, Rb2):
    N = x.shape[0]
    src = edge_index[0].astype(jnp.int32)
    dst = edge_index[1].astype(jnp.int32)

    x_pad = jnp.concatenate([x.astype(jnp.int32), jnp.zeros((240,), jnp.int32)])
    h = _sc_gather(node_emb, x_pad)[:N]
    for l in range(_L):
        hs = _sc_gather(h, src)
        m = hs + jnp.take(edge_embs[l], e, axis=0)
        agg = jax.ops.segment_sum(m, dst, num_segments=N)
        z = agg @ W1s[l] + b1s[l]
        z = jax.nn.relu(z)
        z = z @ W2s[l] + b2s[l]
        mean = jnp.mean(z, axis=0)
        var = jnp.var(z, axis=0)
        z = (z - mean) / jnp.sqrt(var + 1e-5) * gammas[l] + betas[l]
        if l < _L - 1:
            z = jax.nn.relu(z)
        h = z

    hg = jnp.mean(h, axis=0, keepdims=True)
    y = jax.nn.relu(hg @ RW0 + Rb0)
    y = jax.nn.relu(y @ RW1 + Rb1)
    scores = y @ RW2 + Rb2
    return hg, scores


# speedup vs baseline: 1.6142x; 1.3389x over previous
import functools

import jax
import jax.numpy as jnp
from jax import lax
from jax.experimental import pallas as pl
from jax.experimental.pallas import tpu as pltpu
from jax.experimental.pallas import tpu_sc as plsc

_L = 5
_D = 128
_NW = 32
_GC = 400


def _make_sc_gather(B, V):
    b_per_w = B // _NW
    chunk = b_per_w if b_per_w <= 512 else _GC
    niter = b_per_w // chunk
    mesh = plsc.VectorSubcoreMesh(core_axis_name="c", subcore_axis_name="s")

    @functools.partial(
        pl.kernel,
        mesh=mesh,
        out_type=jax.ShapeDtypeStruct((B, _D), jnp.float32),
        scratch_types=[
            pltpu.VMEM((b_per_w,), jnp.int32),
            pltpu.VMEM((chunk, _D), jnp.float32),
            pltpu.SemaphoreType.DMA,
        ],
    )
    def k(table_hbm, idx_hbm, out_hbm, idx_v, rows_v, gsem):
        wid = lax.axis_index("s") * 2 + lax.axis_index("c")
        base = wid * b_per_w
        pltpu.sync_copy(idx_hbm.at[pl.ds(base, b_per_w)], idx_v)

        def step(i, carry):
            pltpu.async_copy(
                table_hbm.at[idx_v.at[pl.ds(i * chunk, chunk)]], rows_v, gsem
            ).wait()
            pltpu.sync_copy(rows_v, out_hbm.at[pl.ds(base + i * chunk, chunk)])
            return carry

        lax.fori_loop(0, niter, step, 0, unroll=False)

    return k


def _sc_gather(table, idx):
    return _make_sc_gather(idx.shape[0], table.shape[0])(table, idx)


def _add_ee_chunk(rows_buf, eidx_v, etab_v, chunk_base, chunk):
    def g_body(g, carry):
        ev = eidx_v[pl.ds(chunk_base + g * 16, 16)]
        for jj in range(16):
            ej = ev[jj]
            row = g * 16 + jj
            for cg in range(_D // 16):
                sl = pl.ds(cg * 16, 16)
                rows_buf[row, sl] = rows_buf[row, sl] + etab_v[ej, sl]
        return carry

    lax.fori_loop(0, chunk // 16, g_body, 0, unroll=False)


def _make_sc_gather_add(B):
    b_per_w = B // _NW
    chunk = _GC
    niter = b_per_w // chunk
    npair = niter // 2
    mesh = plsc.VectorSubcoreMesh(core_axis_name="c", subcore_axis_name="s")

    @functools.partial(
        pl.kernel,
        mesh=mesh,
        out_type=jax.ShapeDtypeStruct((B, _D), jnp.float32),
        scratch_types=[
            pltpu.VMEM((b_per_w,), jnp.int32),
            pltpu.VMEM((b_per_w,), jnp.int32),
            pltpu.VMEM((8, _D), jnp.float32),
            pltpu.VMEM((chunk, _D), jnp.float32),
            pltpu.VMEM((chunk, _D), jnp.float32),
            pltpu.SemaphoreType.DMA,
            pltpu.SemaphoreType.DMA,
        ],
    )
    def k(table_hbm, sidx_hbm, eidx_hbm, etab_hbm, out_hbm,
          sidx_v, eidx_v, etab_v, rows_a, rows_b, sem_a, sem_b):
        wid = lax.axis_index("s") * 2 + lax.axis_index("c")
        base = wid * b_per_w
        pltpu.sync_copy(sidx_hbm.at[pl.ds(base, b_per_w)], sidx_v)
        pltpu.sync_copy(eidx_hbm.at[pl.ds(base, b_per_w)], eidx_v)
        pltpu.sync_copy(etab_hbm, etab_v)

        def gather(ci, buf, sem):
            return pltpu.make_async_copy(
                table_hbm.at[sidx_v.at[pl.ds(ci * chunk, chunk)]], buf, sem
            )

        def finish(ci, buf):
            _add_ee_chunk(buf, eidx_v, etab_v, ci * chunk, chunk)
            pltpu.sync_copy(buf, out_hbm.at[pl.ds(base + ci * chunk, chunk)])

        gather(0, rows_a, sem_a).start()

        def pair(i2, carry):
            c0 = 2 * i2
            gather(c0, rows_a, sem_a).wait()
            gather(c0 + 1, rows_b, sem_b).start()
            finish(c0, rows_a)
            gather(c0 + 1, rows_b, sem_b).wait()

            @pl.when(c0 + 2 < niter)
            def _():
                gather(c0 + 2, rows_a, sem_a).start()

            finish(c0 + 1, rows_b)
            return carry

        lax.fori_loop(0, npair, pair, 0, unroll=False)
        if niter % 2 == 1:
            gather(niter - 1, rows_a, sem_a).wait()
            finish(niter - 1, rows_a)

    return k


def _sc_gather_add(h, src, e, etab):
    etab8 = jnp.zeros((8, _D), jnp.float32).at[:6].set(etab)
    return _make_sc_gather_add(src.shape[0])(h, src, e, etab8)


def kernel(x, e, edge_index, node_emb, edge_embs, W1s, b1s, W2s, b2s, gammas, betas,
           RW0, Rb0, RW1, Rb1, RW2, Rb2):
    N = x.shape[0]
    src = edge_index[0].astype(jnp.int32)
    dst = edge_index[1].astype(jnp.int32)

    x_pad = jnp.concatenate([x.astype(jnp.int32), jnp.zeros((240,), jnp.int32)])
    h = _sc_gather(node_emb, x_pad)[:N]
    e32 = e.astype(jnp.int32)
    for l in range(_L):
        m = _sc_gather_add(h, src, e32, edge_embs[l])
        agg = jax.ops.segment_sum(m, dst, num_segments=N)
        z = agg @ W1s[l] + b1s[l]
        z = jax.nn.relu(z)
        z = z @ W2s[l] + b2s[l]
        mean = jnp.mean(z, axis=0)
        var = jnp.var(z, axis=0)
        z = (z - mean) / jnp.sqrt(var + 1e-5) * gammas[l] + betas[l]
        if l < _L - 1:
            z = jax.nn.relu(z)
        h = z

    hg = jnp.mean(h, axis=0, keepdims=True)
    y = jax.nn.relu(hg @ RW0 + Rb0)
    y = jax.nn.relu(y @ RW1 + Rb1)
    scores = y @ RW2 + Rb2
    return hg, scores


# speedup vs baseline: 1.6146x; 1.0002x over previous
import functools

import jax
import jax.numpy as jnp
from jax import lax
from jax.experimental import pallas as pl
from jax.experimental.pallas import tpu as pltpu
from jax.experimental.pallas import tpu_sc as plsc

_L = 5
_D = 128
_NW = 32
_GC = 400


def _make_sc_gather(B, V):
    b_per_w = B // _NW
    chunk = b_per_w if b_per_w <= 512 else _GC
    niter = b_per_w // chunk
    mesh = plsc.VectorSubcoreMesh(core_axis_name="c", subcore_axis_name="s")

    @functools.partial(
        pl.kernel,
        mesh=mesh,
        out_type=jax.ShapeDtypeStruct((B, _D), jnp.float32),
        scratch_types=[
            pltpu.VMEM((b_per_w,), jnp.int32),
            pltpu.VMEM((chunk, _D), jnp.float32),
            pltpu.SemaphoreType.DMA,
        ],
    )
    def k(table_hbm, idx_hbm, out_hbm, idx_v, rows_v, gsem):
        wid = lax.axis_index("s") * 2 + lax.axis_index("c")
        base = wid * b_per_w
        pltpu.sync_copy(idx_hbm.at[pl.ds(base, b_per_w)], idx_v)

        def step(i, carry):
            pltpu.async_copy(
                table_hbm.at[idx_v.at[pl.ds(i * chunk, chunk)]], rows_v, gsem
            ).wait()
            pltpu.sync_copy(rows_v, out_hbm.at[pl.ds(base + i * chunk, chunk)])
            return carry

        lax.fori_loop(0, niter, step, 0, unroll=False)

    return k


def _sc_gather(table, idx):
    return _make_sc_gather(idx.shape[0], table.shape[0])(table, idx)


def _add_ee_chunk(rows_buf, eidx_v, etab_v, chunk_base, chunk):
    def g_body(g, carry):
        ev = eidx_v[pl.ds(chunk_base + g * 16, 16)]
        for jj in range(16):
            ej = ev[jj]
            row = g * 16 + jj
            for cg in range(_D // 16):
                sl = pl.ds(cg * 16, 16)
                rows_buf[row, sl] = rows_buf[row, sl] + etab_v[ej, sl]
        return carry

    lax.fori_loop(0, chunk // 16, g_body, 0, unroll=False)


def _make_sc_gather_add(B, V):
    b_per_w = B // _NW
    chunk = _GC
    niter = b_per_w // chunk
    npair = niter // 2
    mesh = plsc.VectorSubcoreMesh(core_axis_name="c", subcore_axis_name="s")

    @functools.partial(
        pl.kernel,
        mesh=mesh,
        out_type=jax.ShapeDtypeStruct((B, _D), jnp.float32),
        scratch_types=[
            pltpu.VMEM((b_per_w,), jnp.int32),
            pltpu.VMEM((b_per_w,), jnp.int32),
            pltpu.VMEM((8, _D), jnp.float32),
            pltpu.VMEM((chunk, _D), jnp.float32),
            pltpu.VMEM((chunk, _D), jnp.float32),
            pltpu.SemaphoreType.DMA,
            pltpu.SemaphoreType.DMA,
        ],
    )
    def k(table_hbm, sidx_hbm, eidx_hbm, etab_hbm, out_hbm,
          sidx_v, eidx_v, etab_v, rows_a, rows_b, sem_a, sem_b):
        wid = lax.axis_index("s") * 2 + lax.axis_index("c")
        base = wid * b_per_w
        pltpu.sync_copy(sidx_hbm.at[pl.ds(base, b_per_w)], sidx_v)
        pltpu.sync_copy(eidx_hbm.at[pl.ds(base, b_per_w)], eidx_v)
        pltpu.sync_copy(etab_hbm, etab_v)

        def gather(ci, buf, sem):
            return pltpu.make_async_copy(
                table_hbm.at[sidx_v.at[pl.ds(ci * chunk, chunk)]], buf, sem
            )

        def finish(ci, buf):
            _add_ee_chunk(buf, eidx_v, etab_v, ci * chunk, chunk)
            pltpu.sync_copy(buf, out_hbm.at[pl.ds(base + ci * chunk, chunk)])

        gather(0, rows_a, sem_a).start()

        def pair(i2, carry):
            c0 = 2 * i2
            gather(c0, rows_a, sem_a).wait()
            gather(c0 + 1, rows_b, sem_b).start()
            finish(c0, rows_a)
            gather(c0 + 1, rows_b, sem_b).wait()

            @pl.when(c0 + 2 < niter)
            def _():
                gather(c0 + 2, rows_a, sem_a).start()

            finish(c0 + 1, rows_b)
            return carry

        lax.fori_loop(0, npair, pair, 0, unroll=False)
        if niter % 2 == 1:
            gather(niter - 1, rows_a, sem_a).wait()
            finish(niter - 1, rows_a)

    return k


def _sc_gather_add(h, src, e, etab):
    etab8 = jnp.zeros((8, _D), jnp.float32).at[:6].set(etab)
    return _make_sc_gather_add(src.shape[0], h.shape[0])(h, src, e, etab8)


def kernel(x, e, edge_index, node_emb, edge_embs, W1s, b1s, W2s, b2s, gammas, betas,
           RW0, Rb0, RW1, Rb1, RW2, Rb2):
    N = x.shape[0]
    src = edge_index[0].astype(jnp.int32)
    dst = edge_index[1].astype(jnp.int32)

    x_pad = jnp.concatenate([x.astype(jnp.int32), jnp.zeros((240,), jnp.int32)])
    h = _sc_gather(node_emb, x_pad)[:N]
    e32 = e.astype(jnp.int32)
    for l in range(_L):
        m = _sc_gather_add(h, src, e32, edge_embs[l])
        agg = jax.ops.segment_sum(m, dst, num_segments=N)
        z = agg @ W1s[l] + b1s[l]
        z = jax.nn.relu(z)
        z = z @ W2s[l] + b2s[l]
        mean = jnp.mean(z, axis=0)
        var = jnp.var(z, axis=0)
        z = (z - mean) / jnp.sqrt(var + 1e-5) * gammas[l] + betas[l]
        if l < _L - 1:
            z = jax.nn.relu(z)
        h = z

    hg = jnp.mean(h, axis=0, keepdims=True)
    y = jax.nn.relu(hg @ RW0 + Rb0)
    y = jax.nn.relu(y @ RW1 + Rb1)
    scores = y @ RW2 + Rb2
    return hg, scores
